# Initial kernel scaffold; baseline (speedup 1.0000x reference)
import jax
import jax.numpy as jnp
from jax.experimental import pallas as pl


def kernel(x, fnum, table):
    raise NotImplementedError("write your pallas kernel here")


# speedup vs baseline: 1.0288x; 1.0288x over previous
import functools

import jax
import jax.numpy as jnp
from jax import lax
from jax.experimental import pallas as pl
from jax.experimental.pallas import tpu as pltpu
from jax.experimental.pallas import tpu_sc as plsc

_EMB_LEVELS = 1000000
_EMB_DIM = 16
_L = 16
_SEG = 128

_C1 = -1640531535
_C2 = 0x45D9F3B


def _hash_vec(xv):
    h = xv * jnp.int32(_C1)
    h = h ^ lax.shift_right_logical(h, 16)
    h = h * jnp.int32(_C2)
    h = h ^ lax.shift_right_logical(h, 16)
    q = lax.shift_right_logical(h, 1)
    r0 = h & jnp.int32(1)
    m = jnp.int32(_EMB_LEVELS)
    return lax.rem(lax.rem(q, m) * jnp.int32(2) + r0, m)


def _body(chunk, n_chunks, x2_hbm, seeds_hbm, table_hbm, out_hbm,
          seeds_v, x_v, idx_v, rows_v, sem):
    info = plsc.get_sparse_core_info()
    nc = info.num_cores
    wid = lax.axis_index("s") * nc + lax.axis_index("c")
    per_w = chunk * n_chunks
    n_seg = chunk // _SEG

    pltpu.sync_copy(seeds_hbm, seeds_v)
    seeds_alt = seeds_v[...]
    m = jnp.int32(_EMB_LEVELS)

    def do_chunk(c, _):
        base = wid * per_w + c * chunk
        pltpu.sync_copy(x2_hbm.at[pl.ds(base, chunk)], x_v)

        def hash_group(g, _):
            xv = x_v[pl.ds(g * _L, _L)]
            b = _hash_vec(xv)
            idx_v[pl.ds(g * _L, _L)] = lax.rem(b + seeds_alt, m)
            return _

        lax.fori_loop(0, chunk // _L, hash_group, None, unroll=4)

        handles = [
            pltpu.async_copy(
                table_hbm.at[idx_v.at[pl.ds(k * _SEG, _SEG)]],
                rows_v.at[pl.ds(k * _SEG, _SEG)],
                sem)
            for k in range(n_seg)
        ]
        for h in handles:
            h.wait()
        pltpu.sync_copy(rows_v, out_hbm.at[pl.ds(base, chunk)])
        return _

    lax.fori_loop(0, n_chunks, do_chunk, None)


def kernel(x, fnum, table):
    batch, fields = x.shape
    n = batch * fields
    x2 = jnp.repeat(x.reshape(n), 2)
    seeds = jnp.tile(fnum.astype(jnp.int32), _L // 2)

    info = plsc.get_sparse_core_info()
    nw = info.num_cores * info.num_subcores
    per_w = (2 * n) // nw
    assert per_w * nw == 2 * n
    chunk = 2048
    n_chunks = per_w // chunk
    assert n_chunks * chunk == per_w
    n_seg = chunk // _SEG

    mesh = plsc.VectorSubcoreMesh(core_axis_name="c", subcore_axis_name="s")
    kfn = pl.kernel(
        functools.partial(_body, chunk, n_chunks),
        out_type=jax.ShapeDtypeStruct((2 * n, _EMB_DIM), jnp.float32),
        mesh=mesh,
        compiler_params=pltpu.CompilerParams(use_tc_tiling_on_sc=False),
        scratch_types=[
            pltpu.VMEM((_L,), jnp.int32),
            pltpu.VMEM((chunk,), jnp.int32),
            pltpu.VMEM((chunk,), jnp.int32),
            pltpu.VMEM((chunk, _EMB_DIM), jnp.float32),
            pltpu.SemaphoreType.DMA,
        ],
    )
    out = kfn(x2, seeds, table)
    return out.reshape(batch, fields, 2 * _EMB_DIM)


# speedup vs baseline: 1.2839x; 1.2479x over previous
import functools

import jax
import jax.numpy as jnp
from jax import lax
from jax.experimental import pallas as pl
from jax.experimental.pallas import tpu as pltpu
from jax.experimental.pallas import tpu_sc as plsc

_EMB_LEVELS = 1000000
_EMB_DIM = 16
_L = 16
_SEG = 128

_C1 = -1640531535
_C2 = 0x45D9F3B


def _hash_vec(xv):
    h = xv * jnp.int32(_C1)
    h = h ^ lax.shift_right_logical(h, 16)
    h = h * jnp.int32(_C2)
    h = h ^ lax.shift_right_logical(h, 16)
    q = lax.shift_right_logical(h, 1)
    r0 = h & jnp.int32(1)
    m = jnp.int32(_EMB_LEVELS)
    return lax.rem(lax.rem(q, m) * jnp.int32(2) + r0, m)


def _body(chunk, n_chunks, x_hbm, seeds_hbm, table_hbm, out_hbm,
          seeds_v, x_v, idx0_v, idx1_v, rows0_v, rows1_v, sem, osem):
    info = plsc.get_sparse_core_info()
    nc = info.num_cores
    wid = lax.axis_index("s") * nc + lax.axis_index("c")
    per_w = chunk * n_chunks
    n_seg = chunk // _SEG

    pltpu.sync_copy(seeds_hbm, seeds_v)
    s0 = seeds_v[0, :]
    s1 = seeds_v[1, :]
    m = jnp.int32(_EMB_LEVELS)

    def do_chunk(c, _):
        base = wid * per_w + c * chunk
        pltpu.sync_copy(x_hbm.at[pl.ds(base, chunk)], x_v)

        def hash_group(g, _):
            xv = x_v[pl.ds(g * _L, _L)]
            b = _hash_vec(xv)
            idx0_v[pl.ds(g * _L, _L)] = lax.rem(b + s0, m)
            idx1_v[pl.ds(g * _L, _L)] = lax.rem(b + s1, m)
            return _

        lax.fori_loop(0, chunk // _L, hash_group, None, unroll=4)

        handles = []
        for k in range(n_seg):
            handles.append(pltpu.async_copy(
                table_hbm.at[idx0_v.at[pl.ds(k * _SEG, _SEG)]],
                rows0_v.at[pl.ds(k * _SEG, _SEG)], sem))
            handles.append(pltpu.async_copy(
                table_hbm.at[idx1_v.at[pl.ds(k * _SEG, _SEG)]],
                rows1_v.at[pl.ds(k * _SEG, _SEG)], sem))
        for h in handles:
            h.wait()
        w0 = pltpu.async_copy(rows0_v, out_hbm.at[pl.ds(base, chunk), 0], osem)
        w1 = pltpu.async_copy(rows1_v, out_hbm.at[pl.ds(base, chunk), 1], osem)
        w0.wait()
        w1.wait()
        return _

    lax.fori_loop(0, n_chunks, do_chunk, None)


def kernel(x, fnum, table):
    batch, fields = x.shape
    n = batch * fields
    x_flat = x.reshape(n)
    seeds = jnp.broadcast_to(fnum.reshape(2, 1), (2, _L)).astype(jnp.int32)

    info = plsc.get_sparse_core_info()
    nw = info.num_cores * info.num_subcores
    per_w = n // nw
    assert per_w * nw == n
    chunk = 1664
    n_chunks = per_w // chunk
    assert n_chunks * chunk == per_w

    mesh = plsc.VectorSubcoreMesh(core_axis_name="c", subcore_axis_name="s")
    kfn = pl.kernel(
        functools.partial(_body, chunk, n_chunks),
        out_type=jax.ShapeDtypeStruct((n, 2, _EMB_DIM), jnp.float32),
        mesh=mesh,
        compiler_params=pltpu.CompilerParams(use_tc_tiling_on_sc=False),
        scratch_types=[
            pltpu.VMEM((2, _L), jnp.int32),
            pltpu.VMEM((chunk,), jnp.int32),
            pltpu.VMEM((chunk,), jnp.int32),
            pltpu.VMEM((chunk,), jnp.int32),
            pltpu.VMEM((chunk, _EMB_DIM), jnp.float32),
            pltpu.VMEM((chunk, _EMB_DIM), jnp.float32),
            pltpu.SemaphoreType.DMA,
            pltpu.SemaphoreType.DMA,
        ],
    )
    out = kfn(x_flat, seeds, table)
    return out.reshape(batch, fields, 2 * _EMB_DIM)
